# baseline (speedup 1.0000x reference)
import functools

import jax
import jax.numpy as jnp
from jax import lax
from jax.experimental import pallas as pl
from jax.experimental.pallas import tpu as pltpu
from jax.experimental.pallas import tpu_sc as plsc

N = 10000
NPAD = 10112
RPT = 632
F = 256
D = 128
S = 1000
SPAD = 1024

NC = 2
NS = 16
NW = NC * NS
CHUNK = 125
CT = 40
ISEG = 2
ZROWS = 120


def _mm_body(x_ref, w_ref, o_ref):
    o_ref[...] = jnp.dot(x_ref[...], w_ref[...],
                         preferred_element_type=jnp.float32)


def _feature_matmul(features, W):
    blk = 2000
    return pl.pallas_call(
        _mm_body,
        out_shape=jax.ShapeDtypeStruct((N, D), jnp.float32),
        grid=(N // blk,),
        in_specs=[pl.BlockSpec((blk, F), lambda i: (i, 0)),
                  pl.BlockSpec((F, D), lambda i: (0, 0))],
        out_specs=pl.BlockSpec((blk, D), lambda i: (i, 0)),
    )(features, W)


def _sc_segment_sum(h, idx2, samp):
    n_half = idx2.shape[0] // 2
    assert n_half == NW * CT
    sp_per_tile = SPAD // NS

    mesh = plsc.VectorSubcoreMesh(core_axis_name="c", subcore_axis_name="s")

    @functools.partial(
        pl.kernel,
        out_type=(jax.ShapeDtypeStruct((NC * NPAD, D), jnp.float32),
                  jax.ShapeDtypeStruct((NC * SPAD, D), jnp.float32)),
        mesh=mesh,
        scratch_types=[
            pltpu.VMEM((ISEG, CHUNK), jnp.int32),
            pltpu.VMEM((ISEG, CHUNK), jnp.int32),
            pltpu.VMEM((ISEG, CHUNK), jnp.int32),
            pltpu.VMEM((ISEG, CHUNK), jnp.int32),
            pltpu.VMEM((CHUNK, D), jnp.float32),
            pltpu.VMEM((CHUNK, D), jnp.float32),
            pltpu.VMEM((CHUNK, D), jnp.float32),
            pltpu.VMEM((sp_per_tile,), jnp.int32),
            pltpu.VMEM_SHARED((N, D), jnp.float32),
            pltpu.SemaphoreType.DMA,
            pltpu.SemaphoreType.DMA,
            pltpu.SemaphoreType.DMA,
            pltpu.SemaphoreType.DMA,
            pltpu.SemaphoreType.DMA,
            pltpu.SemaphoreType.DMA,
            pltpu.SemaphoreType.DMA,
            pltpu.SemaphoreType.DMA,
            pltpu.SemaphoreType.DMA,
        ],
    )
    def seg_kernel(h_hbm, idx_hbm, samp_hbm,
                   p_hbm, zs_hbm,
                   srcb0, srcb1, dstb0, dstb1, rows0, rows1, rows2, sidx_v,
                   accum,
                   gsem0, gsem1, gsem2, ssem0, ssem1, ssem2,
                   isem0, isem1, samsem):
        c = lax.axis_index("c")
        s = lax.axis_index("s")
        wid = s * NC + c

        srcb = (srcb0, srcb1)
        dstb = (dstb0, dstb1)
        isem = (isem0, isem1)
        rows = (rows0, rows1, rows2)
        gsem = (gsem0, gsem1, gsem2)
        ssem = (ssem0, ssem1, ssem2)

        z16 = jnp.zeros((16,), jnp.float32)

        @pl.loop(0, ZROWS)
        def _zrow(r):
            for k in range(D // 16):
                rows0[r, pl.ds(k * 16, 16)] = z16

        for j in range(4):
            pltpu.sync_copy(
                rows0.at[pl.ds(0, ZROWS)],
                accum.at[pl.ds(s * RPT + j * ZROWS, ZROWS)])

        @pl.when(s < NS - 1)
        def _():
            pltpu.sync_copy(rows0.at[pl.ds(0, ZROWS)],
                            accum.at[pl.ds(s * RPT + 480, ZROWS)])
            pltpu.sync_copy(rows0.at[pl.ds(0, 32)],
                            accum.at[pl.ds(s * RPT + 600, 32)])

        @pl.when(s == NS - 1)
        def _():
            pltpu.sync_copy(rows0.at[pl.ds(0, 40)],
                            accum.at[pl.ds(s * RPT + 480, 40)])

        nseg = CT // ISEG

        def stage(seg):
            sl = seg % 2
            pltpu.async_copy(idx_hbm.at[pl.ds(wid * CT + seg * ISEG, ISEG)],
                             srcb[sl], isem[sl])
            pltpu.async_copy(
                idx_hbm.at[pl.ds(n_half + wid * CT + seg * ISEG, ISEG)],
                dstb[sl], isem[sl])

        def stage_wait(seg):
            sl = seg % 2
            pltpu.make_async_copy(
                idx_hbm.at[pl.ds(wid * CT + seg * ISEG, ISEG)],
                srcb[sl], isem[sl]).wait()
            pltpu.make_async_copy(
                idx_hbm.at[pl.ds(n_half + wid * CT + seg * ISEG, ISEG)],
                dstb[sl], isem[sl]).wait()

        def g_start(j):
            sl, r = divmod(j, ISEG)
            pltpu.async_copy(h_hbm.at[srcb[(sl) % 2].at[r]], rows[j % 3],
                             gsem[j % 3])

        def g_wait(j):
            sl, r = divmod(j, ISEG)
            pltpu.make_async_copy(h_hbm.at[srcb[(sl) % 2].at[r]],
                                  rows[j % 3], gsem[j % 3]).wait()

        def s_start(j):
            sl, r = divmod(j, ISEG)
            pltpu.async_copy(rows[j % 3], accum.at[dstb[(sl) % 2].at[r]],
                             ssem[j % 3], add=True)

        def s_wait(j):
            sl, r = divmod(j, ISEG)
            pltpu.make_async_copy(rows[j % 3], accum.at[dstb[(sl) % 2].at[r]],
                                  ssem[j % 3]).wait()

        stage(0)
        stage(1)
        plsc.subcore_barrier()
        stage_wait(0)
        g_start(0)

        for j in range(CT):
            g_wait(j)
            s_start(j)
            if j >= 2:
                s_wait(j - 2)
            if (j % ISEG == 1 and j >= ISEG + 1
                    and (j - ISEG - 1) // ISEG + 2 < nseg):
                stage((j - ISEG - 1) // ISEG + 2)
            if j + 1 < CT:
                if (j + 1) % ISEG == 0:
                    stage_wait((j + 1) // ISEG)
                g_start(j + 1)

        s_wait(CT - 2)
        s_wait(CT - 1)
        plsc.subcore_barrier()

        wcp = pltpu.async_copy(
            accum.at[pl.ds(s * RPT, 520)],
            p_hbm.at[pl.ds(c * NPAD + s * RPT, 520)], gsem0)

        @pl.when(s < NS - 1)
        def _():
            pltpu.async_copy(
                accum.at[pl.ds(s * RPT + 520, RPT - 520)],
                p_hbm.at[pl.ds(c * NPAD + s * RPT + 520, RPT - 520)], gsem1)

        pltpu.sync_copy(samp_hbm.at[pl.ds(s * sp_per_tile, sp_per_tile)],
                        sidx_v)
        srows = rows0.at[pl.ds(0, sp_per_tile)]
        pltpu.async_copy(accum.at[sidx_v], srows, samsem).wait()
        pltpu.sync_copy(
            srows,
            zs_hbm.at[pl.ds(c * SPAD + s * sp_per_tile, sp_per_tile)])
        wcp.wait()

        @pl.when(s < NS - 1)
        def _():
            pltpu.make_async_copy(
                accum.at[pl.ds(s * RPT + 520, RPT - 520)],
                p_hbm.at[pl.ds(c * NPAD + s * RPT + 520, RPT - 520)],
                gsem1).wait()

    return seg_kernel(h, idx2, samp)


def _dec_body(p_ref, zs_ref, zm_ref, ip_ref, dist_ref):
    p = p_ref[...]
    zm_ref[...] = p[:N] + p[NPAD:NPAD + N]
    zs = zs_ref[...]
    z = zs[:S] + zs[SPAD:SPAD + S]
    ip = lax.dot_general(z, z, (((1,), (1,)), ((), ())),
                         preferred_element_type=jnp.float32)
    ip_ref[...] = ip
    sq = jnp.sum(z * z, axis=1)
    dist_ref[...] = sq[:, None] - 2.0 * ip + sq[None, :]


def _decoder(p_flat, zs_flat):
    return pl.pallas_call(
        _dec_body,
        out_shape=(jax.ShapeDtypeStruct((N, D), jnp.float32),
                   jax.ShapeDtypeStruct((S, S), jnp.float32),
                   jax.ShapeDtypeStruct((S, S), jnp.float32)),
    )(p_flat, zs_flat)


def kernel(features, edge_index, sampled_nodes, W):
    e = edge_index.shape[1]
    assert e == NW * CT * CHUNK
    idx2 = edge_index.astype(jnp.int32).reshape(2 * (e // CHUNK), CHUNK)
    samp = sampled_nodes.astype(jnp.int32)
    samp_p = jnp.concatenate([samp, jnp.zeros((SPAD - S,), jnp.int32)])

    h = _feature_matmul(features, W)
    p_flat, zs_flat = _sc_segment_sum(h, idx2, samp_p)
    z_mean, ip, dist = _decoder(p_flat, zs_flat)
    return z_mean, ip.reshape(-1), dist.reshape(-1)


# speedup vs baseline: 1.0729x; 1.0729x over previous
import functools

import jax
import jax.numpy as jnp
from jax import lax
from jax.experimental import pallas as pl
from jax.experimental.pallas import tpu as pltpu
from jax.experimental.pallas import tpu_sc as plsc

N = 10000
NPAD = 10112
RPT = 632
F = 256
D = 128
S = 1000
SPAD = 1024

NC = 2
NS = 16
NW = NC * NS
CHUNK = 125
CT = 40
ISEG = 2
ZROWS = 120


def _mm_body(x_ref, w_ref, o_ref):
    o_ref[...] = jnp.dot(x_ref[...], w_ref[...],
                         preferred_element_type=jnp.float32)


def _feature_matmul(features, W):
    blk = 2000
    return pl.pallas_call(
        _mm_body,
        out_shape=jax.ShapeDtypeStruct((N, D), jnp.float32),
        grid=(N // blk,),
        in_specs=[pl.BlockSpec((blk, F), lambda i: (i, 0)),
                  pl.BlockSpec((F, D), lambda i: (0, 0))],
        out_specs=pl.BlockSpec((blk, D), lambda i: (i, 0)),
    )(features, W)


def _sc_segment_sum(h, idx2, samp):
    n_half = idx2.shape[0] // 2
    assert n_half == NW * CT
    sp_per_tile = SPAD // NS

    mesh = plsc.VectorSubcoreMesh(core_axis_name="c", subcore_axis_name="s")

    @functools.partial(
        pl.kernel,
        out_type=(jax.ShapeDtypeStruct((NC * NPAD, D), jnp.float32),
                  jax.ShapeDtypeStruct((NC * SPAD, D), jnp.float32)),
        mesh=mesh,
        scratch_types=[
            pltpu.VMEM((ISEG, CHUNK), jnp.int32),
            pltpu.VMEM((ISEG, CHUNK), jnp.int32),
            pltpu.VMEM((ISEG, CHUNK), jnp.int32),
            pltpu.VMEM((ISEG, CHUNK), jnp.int32),
            pltpu.VMEM((CHUNK, D), jnp.float32),
            pltpu.VMEM((CHUNK, D), jnp.float32),
            pltpu.VMEM((CHUNK, D), jnp.float32),
            pltpu.VMEM((sp_per_tile,), jnp.int32),
            pltpu.VMEM_SHARED((N, D), jnp.float32),
            pltpu.SemaphoreType.DMA,
            pltpu.SemaphoreType.DMA,
            pltpu.SemaphoreType.DMA,
            pltpu.SemaphoreType.DMA,
            pltpu.SemaphoreType.DMA,
            pltpu.SemaphoreType.DMA,
            pltpu.SemaphoreType.DMA,
            pltpu.SemaphoreType.DMA,
            pltpu.SemaphoreType.DMA,
            pltpu.SemaphoreType.DMA,
            pltpu.SemaphoreType.DMA,
        ],
    )
    def seg_kernel(h_hbm, idx_hbm, samp_hbm,
                   p_hbm, zs_hbm,
                   srcb0, srcb1, dstb0, dstb1, rows0, rows1, rows2, sidx_v,
                   accum,
                   gsem0, gsem1, gsem2, ssem0, ssem1, ssem2,
                   isem0, isem1, idem0, idem1, samsem):
        c = lax.axis_index("c")
        s = lax.axis_index("s")
        wid = s * NC + c

        srcb = (srcb0, srcb1)
        dstb = (dstb0, dstb1)
        isem = (isem0, isem1)
        idem = (idem0, idem1)
        rows = (rows0, rows1, rows2)
        gsem = (gsem0, gsem1, gsem2)
        ssem = (ssem0, ssem1, ssem2)

        z16 = jnp.zeros((16,), jnp.float32)

        @pl.loop(0, ZROWS)
        def _zrow(r):
            for k in range(D // 16):
                rows0[r, pl.ds(k * 16, 16)] = z16

        for j in range(4):
            pltpu.sync_copy(
                rows0.at[pl.ds(0, ZROWS)],
                accum.at[pl.ds(s * RPT + j * ZROWS, ZROWS)])

        @pl.when(s < NS - 1)
        def _():
            pltpu.sync_copy(rows0.at[pl.ds(0, ZROWS)],
                            accum.at[pl.ds(s * RPT + 480, ZROWS)])
            pltpu.sync_copy(rows0.at[pl.ds(0, 32)],
                            accum.at[pl.ds(s * RPT + 600, 32)])

        @pl.when(s == NS - 1)
        def _():
            pltpu.sync_copy(rows0.at[pl.ds(0, 40)],
                            accum.at[pl.ds(s * RPT + 480, 40)])

        nseg = CT // ISEG

        def stage_src(seg):
            sl = seg % 2
            pltpu.async_copy(idx_hbm.at[pl.ds(wid * CT + seg * ISEG, ISEG)],
                             srcb[sl], isem[sl])

        def stage_src_wait(seg):
            sl = seg % 2
            pltpu.make_async_copy(
                idx_hbm.at[pl.ds(wid * CT + seg * ISEG, ISEG)],
                srcb[sl], isem[sl]).wait()

        def stage_dst(seg):
            sl = seg % 2
            pltpu.async_copy(
                idx_hbm.at[pl.ds(n_half + wid * CT + seg * ISEG, ISEG)],
                dstb[sl], idem[sl])

        def stage_dst_wait(seg):
            sl = seg % 2
            pltpu.make_async_copy(
                idx_hbm.at[pl.ds(n_half + wid * CT + seg * ISEG, ISEG)],
                dstb[sl], idem[sl]).wait()

        def g_start(j):
            sl, r = divmod(j, ISEG)
            pltpu.async_copy(h_hbm.at[srcb[(sl) % 2].at[r]], rows[j % 3],
                             gsem[j % 3])

        def g_wait(j):
            sl, r = divmod(j, ISEG)
            pltpu.make_async_copy(h_hbm.at[srcb[(sl) % 2].at[r]],
                                  rows[j % 3], gsem[j % 3]).wait()

        def s_start(j):
            sl, r = divmod(j, ISEG)
            pltpu.async_copy(rows[j % 3], accum.at[dstb[(sl) % 2].at[r]],
                             ssem[j % 3], add=True)

        def s_wait(j):
            sl, r = divmod(j, ISEG)
            pltpu.make_async_copy(rows[j % 3], accum.at[dstb[(sl) % 2].at[r]],
                                  ssem[j % 3]).wait()

        assert ISEG == 2
        stage_src(0)
        stage_src(1)
        stage_dst(0)
        stage_dst(1)
        plsc.subcore_barrier()
        stage_src_wait(0)
        g_start(0)

        for j in range(CT):
            if j % 2 == 0:
                stage_dst_wait(j // 2)
            g_wait(j)
            s_start(j)
            if j >= 2:
                s_wait(j - 2)
            if j % 2 == 1:
                if (j - 1) // 2 + 2 < nseg:
                    stage_src((j - 1) // 2 + 2)
                if j >= 3 and (j - 3) // 2 + 2 < nseg:
                    stage_dst((j - 3) // 2 + 2)
            if j + 1 < CT:
                if (j + 1) % 2 == 0:
                    stage_src_wait((j + 1) // 2)
                g_start(j + 1)

        s_wait(CT - 2)
        s_wait(CT - 1)
        plsc.subcore_barrier()

        wcp = pltpu.async_copy(
            accum.at[pl.ds(s * RPT, 520)],
            p_hbm.at[pl.ds(c * NPAD + s * RPT, 520)], gsem0)

        @pl.when(s < NS - 1)
        def _():
            pltpu.async_copy(
                accum.at[pl.ds(s * RPT + 520, RPT - 520)],
                p_hbm.at[pl.ds(c * NPAD + s * RPT + 520, RPT - 520)], gsem1)

        pltpu.sync_copy(samp_hbm.at[pl.ds(s * sp_per_tile, sp_per_tile)],
                        sidx_v)
        srows = rows0.at[pl.ds(0, sp_per_tile)]
        pltpu.async_copy(accum.at[sidx_v], srows, samsem).wait()
        pltpu.sync_copy(
            srows,
            zs_hbm.at[pl.ds(c * SPAD + s * sp_per_tile, sp_per_tile)])
        wcp.wait()

        @pl.when(s < NS - 1)
        def _():
            pltpu.make_async_copy(
                accum.at[pl.ds(s * RPT + 520, RPT - 520)],
                p_hbm.at[pl.ds(c * NPAD + s * RPT + 520, RPT - 520)],
                gsem1).wait()

    return seg_kernel(h, idx2, samp)


def _dec_body(p_ref, zs_ref, zm_ref, ip_ref, dist_ref):
    p = p_ref[...]
    zm_ref[...] = p[:N] + p[NPAD:NPAD + N]
    zs = zs_ref[...]
    z = zs[:S] + zs[SPAD:SPAD + S]
    ip = lax.dot_general(z, z, (((1,), (1,)), ((), ())),
                         preferred_element_type=jnp.float32)
    ip_ref[...] = ip
    sq = jnp.sum(z * z, axis=1)
    dist_ref[...] = sq[:, None] - 2.0 * ip + sq[None, :]


def _decoder(p_flat, zs_flat):
    return pl.pallas_call(
        _dec_body,
        out_shape=(jax.ShapeDtypeStruct((N, D), jnp.float32),
                   jax.ShapeDtypeStruct((S, S), jnp.float32),
                   jax.ShapeDtypeStruct((S, S), jnp.float32)),
    )(p_flat, zs_flat)


def kernel(features, edge_index, sampled_nodes, W):
    e = edge_index.shape[1]
    assert e == NW * CT * CHUNK
    idx2 = edge_index.astype(jnp.int32).reshape(2 * (e // CHUNK), CHUNK)
    samp = sampled_nodes.astype(jnp.int32)
    samp_p = jnp.concatenate([samp, jnp.zeros((SPAD - S,), jnp.int32)])

    h = _feature_matmul(features, W)
    p_flat, zs_flat = _sc_segment_sum(h, idx2, samp_p)
    z_mean, ip, dist = _decoder(p_flat, zs_flat)
    return z_mean, ip.reshape(-1), dist.reshape(-1)
